# baseline (speedup 1.0000x reference)
import functools

import jax
import jax.numpy as jnp
from jax import lax
from jax.experimental import pallas as pl
from jax.experimental.pallas import tpu as pltpu, tpu_sc as plsc

DIM = 32
LANES = 16
TROWS = 8


@functools.lru_cache(maxsize=None)
def _make_sc_kernel(batch: int):
    info = plsc.get_sparse_core_info()
    nc, ns = info.num_cores, info.num_subcores
    nw = nc * ns
    bpw = batch // nw
    assert bpw % LANES == 0
    nchunks = bpw // LANES
    assert nchunks % 2 == 0

    mesh = plsc.VectorSubcoreMesh(core_axis_name="c", subcore_axis_name="s")

    @functools.partial(
        pl.kernel,
        mesh=mesh,
        compiler_params=pltpu.CompilerParams(use_tc_tiling_on_sc=True),
        out_type=jax.ShapeDtypeStruct((nw, bpw // 128, 128), jnp.float32),
        scratch_types=[
            pltpu.VMEM((bpw // 128, 128), jnp.int32),
            pltpu.VMEM((bpw // 128, 128), jnp.int32),
            pltpu.VMEM((bpw // 128, 128), jnp.int32),
            pltpu.VMEM((bpw // 128, 128), jnp.int32),
            pltpu.VMEM((bpw // 128, 128), jnp.int32),
            pltpu.VMEM((bpw // 128, 128), jnp.int32),
            pltpu.VMEM((2, LANES * TROWS, DIM), jnp.float32),
            pltpu.VMEM((2, LANES * TROWS, DIM), jnp.float32),
            pltpu.VMEM((2, LANES * TROWS, DIM), jnp.float32),
            pltpu.VMEM((bpw // 128, 128), jnp.float32),
            pltpu.SemaphoreType.DMA,
            pltpu.SemaphoreType.DMA,
            pltpu.SemaphoreType.DMA,
            pltpu.SemaphoreType.DMA,
            pltpu.SemaphoreType.DMA,
            pltpu.SemaphoreType.DMA,
        ],
    )
    def sc_kernel(h_blk_hbm, r_blk_hbm, t_blk_hbm,
                  h_sub_hbm, r_sub_hbm, t_sub_hbm,
                  ent_hbm, rel_hbm, out_hbm,
                  h_blk_v, r_blk_v, t_blk_v, h_sub_v, r_sub_v, t_sub_v,
                  h_c, r_c, t_c, s_v,
                  sem_h0, sem_h1, sem_r0, sem_r1, sem_t0, sem_t1):
        sems = ((sem_h0, sem_r0, sem_t0), (sem_h1, sem_r1, sem_t1))
        wid = lax.axis_index("s") * nc + lax.axis_index("c")
        crow = wid * (bpw // 128)

        for src, dst in ((h_blk_hbm, h_blk_v), (r_blk_hbm, r_blk_v),
                         (t_blk_hbm, t_blk_v), (h_sub_hbm, h_sub_v),
                         (r_sub_hbm, r_sub_v), (t_sub_hbm, t_sub_v)):
            pltpu.sync_copy(src.at[pl.ds(crow, bpw // 128)], dst)

        def _idx_vec(ref, j):
            return ref[j >> 3, pl.ds((j & 7) * LANES, LANES)]

        def fire(j, b):
            bufs = (h_c, r_c, t_c)
            tabs = (ent_hbm, rel_hbm, ent_hbm)
            vecs = (_idx_vec(h_blk_v, j), _idx_vec(r_blk_v, j),
                    _idx_vec(t_blk_v, j))
            for tbl in range(3):
                bv = vecs[tbl]
                for k in range(LANES):
                    base = pl.multiple_of(bv[k], TROWS)
                    pltpu.async_copy(
                        tabs[tbl].at[pl.ds(base, TROWS)],
                        bufs[tbl].at[b, pl.ds(k * TROWS, TROWS)],
                        sems[b][tbl])

        def drain(b):
            src = ent_hbm.at[pl.ds(0, LANES * TROWS)]
            pltpu.make_async_copy(src, h_c.at[b], sems[b][0]).wait()
            pltpu.make_async_copy(src, r_c.at[b], sems[b][1]).wait()
            pltpu.make_async_copy(src, t_c.at[b], sems[b][2]).wait()

        lane = lax.iota(jnp.int32, LANES)
        _dnums = lax.GatherDimensionNumbers(
            offset_dims=(), collapsed_slice_dims=(0,), start_index_map=(0,))

        def _permute(a, idx):
            return lax.gather(
                a, idx[:, None], _dnums, slice_sizes=(1,),
                mode=lax.GatherScatterMode.PROMISE_IN_BOUNDS)

        bitrev = [0, 8, 4, 12, 2, 10, 6, 14, 1, 9, 5, 13, 3, 11, 7, 15]

        def compute(j, b):
            hs = _idx_vec(h_sub_v, j)
            rs = _idx_vec(r_sub_v, j)
            ts = _idx_vec(t_sub_v, j)
            ws = []
            for k in range(LANES):
                kk = bitrev[k]
                hrow = kk * TROWS + hs[kk]
                rrow = kk * TROWS + rs[kk]
                trow = kk * TROWS + ts[kk]
                d0 = (h_c[b, hrow, pl.ds(0, LANES)]
                      + r_c[b, rrow, pl.ds(0, LANES)]
                      - t_c[b, trow, pl.ds(0, LANES)])
                d1 = (h_c[b, hrow, pl.ds(LANES, LANES)]
                      + r_c[b, rrow, pl.ds(LANES, LANES)]
                      - t_c[b, trow, pl.ds(LANES, LANES)])
                ws.append(d0 * d0 + d1 * d1)
            stride = LANES // 2
            while len(ws) > 1:
                perm = lane ^ stride
                keep = (lane & stride) == 0
                nxt = []
                for a, c in zip(ws[0::2], ws[1::2]):
                    pa = a + _permute(a, perm)
                    pc = c + _permute(c, perm)
                    nxt.append(jnp.where(keep, pa, pc))
                ws = nxt
                stride //= 2
            ssq = ws[0]
            bits = lax.bitcast_convert_type(ssq, jnp.int32)
            y = lax.bitcast_convert_type(
                jnp.int32(0x5F3759DF) - lax.shift_right_logical(bits, 1),
                jnp.float32)
            half = ssq * jnp.float32(0.5)
            for _ in range(3):
                y = y * (jnp.float32(1.5) - half * y * y)
            s_v[j >> 3, pl.ds((j & 7) * LANES, LANES)] = -(ssq * y)

        fire(jnp.int32(0), 0)

        def pair_body(p, carry):
            j0 = p * 2
            j1 = j0 + 1
            fire(j1, 1)
            drain(0)
            compute(j0, 0)

            @pl.when(p < (nchunks // 2 - 1))
            def _():
                fire(j0 + 2, 0)

            drain(1)
            compute(j1, 1)
            return carry

        lax.fori_loop(0, nchunks // 2, pair_body, jnp.int32(0))

        pltpu.sync_copy(s_v, out_hbm.at[wid])

    return sc_kernel


def kernel(data, ent_emb, rel_emb):
    batch = data.shape[0]
    blk = (data // TROWS) * TROWS
    sub = data % TROWS
    h_blk = blk[:, 0].reshape(-1, 128)
    r_blk = blk[:, 1].reshape(-1, 128)
    t_blk = blk[:, 2].reshape(-1, 128)
    h_sub = sub[:, 0].reshape(-1, 128)
    r_sub = sub[:, 1].reshape(-1, 128)
    t_sub = sub[:, 2].reshape(-1, 128)
    k = _make_sc_kernel(batch)
    scores = k(h_blk, r_blk, t_blk, h_sub, r_sub, t_sub, ent_emb, rel_emb)
    return scores.reshape(batch)


# speedup vs baseline: 1.5776x; 1.5776x over previous
import functools

import jax
import jax.numpy as jnp
from jax import lax
from jax.experimental import pallas as pl
from jax.experimental.pallas import tpu as pltpu, tpu_sc as plsc

DIM = 32
LANES = 16
TROWS = 8
COL = 128


def _sc_info():
    info = plsc.get_sparse_core_info()
    return info.num_cores, info.num_subcores


@functools.lru_cache(maxsize=None)
def _make_rel_kernel(batch: int):
    nc, ns = _sc_info()
    nw = nc * ns
    bpw = batch // nw
    npairs = bpw // 2

    mesh = plsc.VectorSubcoreMesh(core_axis_name="c", subcore_axis_name="s")

    @functools.partial(
        pl.kernel,
        mesh=mesh,
        compiler_params=pltpu.CompilerParams(use_tc_tiling_on_sc=True),
        out_type=jax.ShapeDtypeStruct((nw, bpw, DIM), jnp.float32),
        scratch_types=[
            pltpu.VMEM((bpw // 128, 128), jnp.int32),
            pltpu.VMEM((bpw // 128, 128), jnp.int32),
            pltpu.VMEM((2, DIM, COL), jnp.float32),
            pltpu.VMEM((bpw, DIM), jnp.float32),
            pltpu.SemaphoreType.DMA,
            pltpu.SemaphoreType.DMA,
        ],
    )
    def rel_kernel(cb_hbm, ln_hbm, rT_hbm, out_hbm,
                   cb_v, ln_v, blk, rows_v, sem0, sem1):
        sems = (sem0, sem1)
        wid = lax.axis_index("s") * nc + lax.axis_index("c")
        crow = wid * (bpw // 128)
        pltpu.sync_copy(cb_hbm.at[pl.ds(crow, bpw // 128)], cb_v)
        pltpu.sync_copy(ln_hbm.at[pl.ds(crow, bpw // 128)], ln_v)

        lane = lax.iota(jnp.int32, LANES)
        _dnums = lax.GatherDimensionNumbers(
            offset_dims=(), collapsed_slice_dims=(0,), start_index_map=(0,))

        def _permute(a, idx):
            return lax.gather(
                a, idx[:, None], _dnums, slice_sizes=(1,),
                mode=lax.GatherScatterMode.PROMISE_IN_BOUNDS)

        def fire(cb_scal, b):
            cb = pl.multiple_of(cb_scal, COL)
            pltpu.async_copy(rT_hbm.at[pl.ds(0, DIM), pl.ds(cb, COL)],
                             blk.at[b], sems[b])

        def drain(b):
            pltpu.make_async_copy(rT_hbm.at[pl.ds(0, DIM), pl.ds(0, COL)],
                                  blk.at[b], sems[b]).wait()

        def extract(l_scal, e, b):
            seg = (l_scal >> 4) << 4
            sl = jnp.full((LANES,), l_scal & 15, jnp.int32)
            for q in range(2):
                acc = jnp.zeros((LANES,), jnp.float32)
                for p in range(LANES):
                    v = blk[b, q * LANES + p, pl.ds(seg, LANES)]
                    pv = _permute(v, sl)
                    acc = jnp.where(lane == p, pv, acc)
                rows_v[e, pl.ds(q * LANES, LANES)] = acc

        def group_body(g, carry):
            cbv = cb_v[g >> 3, pl.ds((g & 7) * LANES, LANES)]
            lnv = ln_v[g >> 3, pl.ds((g & 7) * LANES, LANES)]
            fire(cbv[0], 0)
            for k in range(LANES):
                if k + 1 < LANES:
                    fire(cbv[k + 1], (k + 1) % 2)
                drain(k % 2)
                extract(lnv[k], g * LANES + k, k % 2)
            return carry

        lax.fori_loop(0, bpw // LANES, group_body, jnp.int32(0))
        pltpu.sync_copy(rows_v, out_hbm.at[wid])

    return rel_kernel


@functools.lru_cache(maxsize=None)
def _make_score_kernel(batch: int):
    nc, ns = _sc_info()
    nw = nc * ns
    bpw = batch // nw
    nchunks = bpw // LANES

    mesh = plsc.VectorSubcoreMesh(core_axis_name="c", subcore_axis_name="s")

    @functools.partial(
        pl.kernel,
        mesh=mesh,
        compiler_params=pltpu.CompilerParams(use_tc_tiling_on_sc=True),
        out_type=jax.ShapeDtypeStruct((nw, bpw // 128, 128), jnp.float32),
        scratch_types=[
            pltpu.VMEM((bpw // 128, 128), jnp.int32),
            pltpu.VMEM((bpw // 128, 128), jnp.int32),
            pltpu.VMEM((bpw // 128, 128), jnp.int32),
            pltpu.VMEM((bpw // 128, 128), jnp.int32),
            pltpu.VMEM((2, LANES * TROWS, DIM), jnp.float32),
            pltpu.VMEM((2, LANES * TROWS, DIM), jnp.float32),
            pltpu.VMEM((2, LANES, DIM), jnp.float32),
            pltpu.VMEM((bpw // 128, 128), jnp.float32),
            pltpu.SemaphoreType.DMA,
            pltpu.SemaphoreType.DMA,
            pltpu.SemaphoreType.DMA,
            pltpu.SemaphoreType.DMA,
            pltpu.SemaphoreType.DMA,
            pltpu.SemaphoreType.DMA,
        ],
    )
    def score_kernel(h_blk_hbm, t_blk_hbm, h_sub_hbm, t_sub_hbm,
                     ent_hbm, rel_rows_hbm, out_hbm,
                     h_blk_v, t_blk_v, h_sub_v, t_sub_v,
                     h_c, t_c, r_c, s_v,
                     sem_h0, sem_h1, sem_t0, sem_t1, sem_r0, sem_r1):
        sems = ((sem_h0, sem_t0, sem_r0), (sem_h1, sem_t1, sem_r1))
        wid = lax.axis_index("s") * nc + lax.axis_index("c")
        crow = wid * (bpw // 128)

        for src, dst in ((h_blk_hbm, h_blk_v), (t_blk_hbm, t_blk_v),
                         (h_sub_hbm, h_sub_v), (t_sub_hbm, t_sub_v)):
            pltpu.sync_copy(src.at[pl.ds(crow, bpw // 128)], dst)

        def _idx_vec(ref, j):
            return ref[j >> 3, pl.ds((j & 7) * LANES, LANES)]

        def fire(j, b):
            hv = _idx_vec(h_blk_v, j)
            tv = _idx_vec(t_blk_v, j)
            for k in range(LANES):
                hb = pl.multiple_of(hv[k], TROWS)
                tb = pl.multiple_of(tv[k], TROWS)
                pltpu.async_copy(ent_hbm.at[pl.ds(hb, TROWS)],
                                 h_c.at[b, pl.ds(k * TROWS, TROWS)],
                                 sems[b][0])
                pltpu.async_copy(ent_hbm.at[pl.ds(tb, TROWS)],
                                 t_c.at[b, pl.ds(k * TROWS, TROWS)],
                                 sems[b][1])
            pltpu.async_copy(rel_rows_hbm.at[wid].at[pl.ds(j * LANES, LANES)],
                             r_c.at[b], sems[b][2])

        def drain(b):
            src = ent_hbm.at[pl.ds(0, LANES * TROWS)]
            pltpu.make_async_copy(src, h_c.at[b], sems[b][0]).wait()
            pltpu.make_async_copy(src, t_c.at[b], sems[b][1]).wait()
            srcr = rel_rows_hbm.at[0].at[pl.ds(0, LANES)]
            pltpu.make_async_copy(srcr, r_c.at[b], sems[b][2]).wait()

        lane = lax.iota(jnp.int32, LANES)
        _dnums = lax.GatherDimensionNumbers(
            offset_dims=(), collapsed_slice_dims=(0,), start_index_map=(0,))

        def _permute(a, idx):
            return lax.gather(
                a, idx[:, None], _dnums, slice_sizes=(1,),
                mode=lax.GatherScatterMode.PROMISE_IN_BOUNDS)

        bitrev = [0, 8, 4, 12, 2, 10, 6, 14, 1, 9, 5, 13, 3, 11, 7, 15]

        def compute(j, b):
            hs = _idx_vec(h_sub_v, j)
            ts = _idx_vec(t_sub_v, j)
            ws = []
            for k in range(LANES):
                kk = bitrev[k]
                hrow = kk * TROWS + hs[kk]
                trow = kk * TROWS + ts[kk]
                d0 = (h_c[b, hrow, pl.ds(0, LANES)]
                      + r_c[b, kk, pl.ds(0, LANES)]
                      - t_c[b, trow, pl.ds(0, LANES)])
                d1 = (h_c[b, hrow, pl.ds(LANES, LANES)]
                      + r_c[b, kk, pl.ds(LANES, LANES)]
                      - t_c[b, trow, pl.ds(LANES, LANES)])
                ws.append(d0 * d0 + d1 * d1)
            stride = LANES // 2
            while len(ws) > 1:
                perm = lane ^ stride
                keep = (lane & stride) == 0
                nxt = []
                for a, c in zip(ws[0::2], ws[1::2]):
                    pa = a + _permute(a, perm)
                    pc = c + _permute(c, perm)
                    nxt.append(jnp.where(keep, pa, pc))
                ws = nxt
                stride //= 2
            ssq = ws[0]
            bits = lax.bitcast_convert_type(ssq, jnp.int32)
            y = lax.bitcast_convert_type(
                jnp.int32(0x5F3759DF) - lax.shift_right_logical(bits, 1),
                jnp.float32)
            half = ssq * jnp.float32(0.5)
            for _ in range(3):
                y = y * (jnp.float32(1.5) - half * y * y)
            s_v[j >> 3, pl.ds((j & 7) * LANES, LANES)] = -(ssq * y)

        fire(jnp.int32(0), 0)

        def pair_body(p, carry):
            j0 = p * 2
            fire(j0 + 1, 1)
            drain(0)
            compute(j0, 0)

            @pl.when(p < (nchunks // 2 - 1))
            def _():
                fire(j0 + 2, 0)

            drain(1)
            compute(j0 + 1, 1)
            return carry

        lax.fori_loop(0, nchunks // 2, pair_body, jnp.int32(0))

        pltpu.sync_copy(s_v, out_hbm.at[wid])

    return score_kernel


def kernel(data, ent_emb, rel_emb):
    batch = data.shape[0]
    ent_tot = ent_emb.shape[0]

    h_idx, r_idx, t_idx = data[:, 0], data[:, 1], data[:, 2]
    r_cb = jnp.minimum((r_idx // COL) * COL, ent_tot - COL)
    r_ln = r_idx - r_cb
    h_blk = (h_idx // TROWS) * TROWS
    t_blk = (t_idx // TROWS) * TROWS
    h_sub = h_idx % TROWS
    t_sub = t_idx % TROWS

    rel_rows = _make_rel_kernel(batch)(
        r_cb.reshape(-1, 128), r_ln.reshape(-1, 128), rel_emb.T)
    scores = _make_score_kernel(batch)(
        h_blk.reshape(-1, 128), t_blk.reshape(-1, 128),
        h_sub.reshape(-1, 128), t_sub.reshape(-1, 128),
        ent_emb, rel_rows)
    return scores.reshape(batch)


# speedup vs baseline: 1.6530x; 1.0478x over previous
import functools

import jax
import jax.numpy as jnp
from jax import lax
from jax.experimental import pallas as pl
from jax.experimental.pallas import tpu as pltpu, tpu_sc as plsc

DIM = 32
LANES = 16
TROWS = 8
COL = 128


def _sc_info():
    info = plsc.get_sparse_core_info()
    return info.num_cores, info.num_subcores


@functools.lru_cache(maxsize=None)
def _make_rel_kernel(batch: int):
    nc, ns = _sc_info()
    nw = nc * ns
    bpw = batch // nw
    npairs = bpw // 2

    mesh = plsc.VectorSubcoreMesh(core_axis_name="c", subcore_axis_name="s")

    @functools.partial(
        pl.kernel,
        mesh=mesh,
        compiler_params=pltpu.CompilerParams(use_tc_tiling_on_sc=True),
        out_type=jax.ShapeDtypeStruct((nw, bpw, DIM), jnp.float32),
        scratch_types=[
            pltpu.VMEM((bpw // 128, 128), jnp.int32),
            pltpu.VMEM((bpw // 128, 128), jnp.int32),
            pltpu.VMEM((4, DIM, COL), jnp.float32),
            pltpu.VMEM((bpw, DIM), jnp.float32),
            pltpu.SemaphoreType.DMA,
            pltpu.SemaphoreType.DMA,
            pltpu.SemaphoreType.DMA,
            pltpu.SemaphoreType.DMA,
        ],
    )
    def rel_kernel(cb_hbm, ln_hbm, rT_hbm, out_hbm,
                   cb_v, ln_v, blk, rows_v, sem0, sem1, sem2, sem3):
        sems = (sem0, sem1, sem2, sem3)
        wid = lax.axis_index("s") * nc + lax.axis_index("c")
        crow = wid * (bpw // 128)
        pltpu.sync_copy(cb_hbm.at[pl.ds(crow, bpw // 128)], cb_v)
        pltpu.sync_copy(ln_hbm.at[pl.ds(crow, bpw // 128)], ln_v)

        lane = lax.iota(jnp.int32, LANES)
        _dnums = lax.GatherDimensionNumbers(
            offset_dims=(), collapsed_slice_dims=(0,), start_index_map=(0,))

        def _permute(a, idx):
            return lax.gather(
                a, idx[:, None], _dnums, slice_sizes=(1,),
                mode=lax.GatherScatterMode.PROMISE_IN_BOUNDS)

        def fire(cb_scal, b):
            cb = pl.multiple_of(cb_scal, COL)
            for a in range(4):
                pltpu.async_copy(
                    rT_hbm.at[pl.ds(a * 8, 8), pl.ds(cb, COL)],
                    blk.at[b, pl.ds(a * 8, 8)], sems[b])

        def drain(b):
            pltpu.make_async_copy(rT_hbm.at[pl.ds(0, DIM), pl.ds(0, COL)],
                                  blk.at[b], sems[b]).wait()

        def extract(l_scal, e, b):
            seg = (l_scal >> 4) << 4
            sl = jnp.full((LANES,), l_scal & 15, jnp.int32)
            for q in range(2):
                acc = jnp.zeros((LANES,), jnp.float32)
                for p in range(LANES):
                    v = blk[b, q * LANES + p, pl.ds(seg, LANES)]
                    pv = _permute(v, sl)
                    acc = jnp.where(lane == p, pv, acc)
                rows_v[e, pl.ds(q * LANES, LANES)] = acc

        def group_body(g, carry):
            cbv = cb_v[g >> 3, pl.ds((g & 7) * LANES, LANES)]
            lnv = ln_v[g >> 3, pl.ds((g & 7) * LANES, LANES)]
            for k in range(3):
                fire(cbv[k], k)
            for k in range(LANES):
                if k + 3 < LANES:
                    fire(cbv[k + 3], (k + 3) % 4)
                drain(k % 4)
                extract(lnv[k], g * LANES + k, k % 4)
            return carry

        lax.fori_loop(0, bpw // LANES, group_body, jnp.int32(0))
        pltpu.sync_copy(rows_v, out_hbm.at[wid])

    return rel_kernel


@functools.lru_cache(maxsize=None)
def _make_score_kernel(batch: int):
    nc, ns = _sc_info()
    nw = nc * ns
    bpw = batch // nw
    nchunks = bpw // LANES

    mesh = plsc.VectorSubcoreMesh(core_axis_name="c", subcore_axis_name="s")

    @functools.partial(
        pl.kernel,
        mesh=mesh,
        compiler_params=pltpu.CompilerParams(use_tc_tiling_on_sc=True),
        out_type=jax.ShapeDtypeStruct((nw, bpw // 128, 128), jnp.float32),
        scratch_types=[
            pltpu.VMEM((bpw // 128, 128), jnp.int32),
            pltpu.VMEM((bpw // 128, 128), jnp.int32),
            pltpu.VMEM((bpw // 128, 128), jnp.int32),
            pltpu.VMEM((bpw // 128, 128), jnp.int32),
            pltpu.VMEM((2, LANES * TROWS, DIM), jnp.float32),
            pltpu.VMEM((2, LANES * TROWS, DIM), jnp.float32),
            pltpu.VMEM((2, LANES, DIM), jnp.float32),
            pltpu.VMEM((bpw // 128, 128), jnp.float32),
            pltpu.SemaphoreType.DMA,
            pltpu.SemaphoreType.DMA,
            pltpu.SemaphoreType.DMA,
            pltpu.SemaphoreType.DMA,
            pltpu.SemaphoreType.DMA,
            pltpu.SemaphoreType.DMA,
        ],
    )
    def score_kernel(h_blk_hbm, t_blk_hbm, h_sub_hbm, t_sub_hbm,
                     ent_hbm, rel_rows_hbm, out_hbm,
                     h_blk_v, t_blk_v, h_sub_v, t_sub_v,
                     h_c, t_c, r_c, s_v,
                     sem_h0, sem_h1, sem_t0, sem_t1, sem_r0, sem_r1):
        sems = ((sem_h0, sem_t0, sem_r0), (sem_h1, sem_t1, sem_r1))
        wid = lax.axis_index("s") * nc + lax.axis_index("c")
        crow = wid * (bpw // 128)

        for src, dst in ((h_blk_hbm, h_blk_v), (t_blk_hbm, t_blk_v),
                         (h_sub_hbm, h_sub_v), (t_sub_hbm, t_sub_v)):
            pltpu.sync_copy(src.at[pl.ds(crow, bpw // 128)], dst)

        def _idx_vec(ref, j):
            return ref[j >> 3, pl.ds((j & 7) * LANES, LANES)]

        def fire(j, b):
            hv = _idx_vec(h_blk_v, j)
            tv = _idx_vec(t_blk_v, j)
            for k in range(LANES):
                hb = pl.multiple_of(hv[k], TROWS)
                tb = pl.multiple_of(tv[k], TROWS)
                pltpu.async_copy(ent_hbm.at[pl.ds(hb, TROWS)],
                                 h_c.at[b, pl.ds(k * TROWS, TROWS)],
                                 sems[b][0])
                pltpu.async_copy(ent_hbm.at[pl.ds(tb, TROWS)],
                                 t_c.at[b, pl.ds(k * TROWS, TROWS)],
                                 sems[b][1])
            pltpu.async_copy(rel_rows_hbm.at[wid].at[pl.ds(j * LANES, LANES)],
                             r_c.at[b], sems[b][2])

        def drain(b):
            src = ent_hbm.at[pl.ds(0, LANES * TROWS)]
            pltpu.make_async_copy(src, h_c.at[b], sems[b][0]).wait()
            pltpu.make_async_copy(src, t_c.at[b], sems[b][1]).wait()
            srcr = rel_rows_hbm.at[0].at[pl.ds(0, LANES)]
            pltpu.make_async_copy(srcr, r_c.at[b], sems[b][2]).wait()

        lane = lax.iota(jnp.int32, LANES)
        _dnums = lax.GatherDimensionNumbers(
            offset_dims=(), collapsed_slice_dims=(0,), start_index_map=(0,))

        def _permute(a, idx):
            return lax.gather(
                a, idx[:, None], _dnums, slice_sizes=(1,),
                mode=lax.GatherScatterMode.PROMISE_IN_BOUNDS)

        bitrev = [0, 8, 4, 12, 2, 10, 6, 14, 1, 9, 5, 13, 3, 11, 7, 15]

        def compute(j, b):
            hs = _idx_vec(h_sub_v, j)
            ts = _idx_vec(t_sub_v, j)
            ws = []
            for k in range(LANES):
                kk = bitrev[k]
                hrow = kk * TROWS + hs[kk]
                trow = kk * TROWS + ts[kk]
                d0 = (h_c[b, hrow, pl.ds(0, LANES)]
                      + r_c[b, kk, pl.ds(0, LANES)]
                      - t_c[b, trow, pl.ds(0, LANES)])
                d1 = (h_c[b, hrow, pl.ds(LANES, LANES)]
                      + r_c[b, kk, pl.ds(LANES, LANES)]
                      - t_c[b, trow, pl.ds(LANES, LANES)])
                ws.append(d0 * d0 + d1 * d1)
            stride = LANES // 2
            while len(ws) > 1:
                perm = lane ^ stride
                keep = (lane & stride) == 0
                nxt = []
                for a, c in zip(ws[0::2], ws[1::2]):
                    pa = a + _permute(a, perm)
                    pc = c + _permute(c, perm)
                    nxt.append(jnp.where(keep, pa, pc))
                ws = nxt
                stride //= 2
            ssq = ws[0]
            bits = lax.bitcast_convert_type(ssq, jnp.int32)
            y = lax.bitcast_convert_type(
                jnp.int32(0x5F3759DF) - lax.shift_right_logical(bits, 1),
                jnp.float32)
            half = ssq * jnp.float32(0.5)
            for _ in range(3):
                y = y * (jnp.float32(1.5) - half * y * y)
            s_v[j >> 3, pl.ds((j & 7) * LANES, LANES)] = -(ssq * y)

        fire(jnp.int32(0), 0)

        def pair_body(p, carry):
            j0 = p * 2
            fire(j0 + 1, 1)
            drain(0)
            compute(j0, 0)

            @pl.when(p < (nchunks // 2 - 1))
            def _():
                fire(j0 + 2, 0)

            drain(1)
            compute(j0 + 1, 1)
            return carry

        lax.fori_loop(0, nchunks // 2, pair_body, jnp.int32(0))

        pltpu.sync_copy(s_v, out_hbm.at[wid])

    return score_kernel


def kernel(data, ent_emb, rel_emb):
    batch = data.shape[0]
    ent_tot = ent_emb.shape[0]

    h_idx, r_idx, t_idx = data[:, 0], data[:, 1], data[:, 2]
    r_cb = jnp.minimum((r_idx // COL) * COL, ent_tot - COL)
    r_ln = r_idx - r_cb
    h_blk = (h_idx // TROWS) * TROWS
    t_blk = (t_idx // TROWS) * TROWS
    h_sub = h_idx % TROWS
    t_sub = t_idx % TROWS

    rel_rows = _make_rel_kernel(batch)(
        r_cb.reshape(-1, 128), r_ln.reshape(-1, 128), rel_emb.T)
    scores = _make_score_kernel(batch)(
        h_blk.reshape(-1, 128), t_blk.reshape(-1, 128),
        h_sub.reshape(-1, 128), t_sub.reshape(-1, 128),
        ent_emb, rel_rows)
    return scores.reshape(batch)
